# baseline (device time: 11768 ns/iter reference)
import jax
import jax.numpy as jnp
from jax import lax
from jax.experimental import pallas as pl
from jax.experimental.pallas import tpu as pltpu

N_DEV = 8


def kernel(x, t_emb, W_scale, W_shift):
    b, s, c_per = x.shape

    def body(x_ref, t_ref, wsc_ref, wsh_ref, out_ref,
             gather_buf, send_sems, recv_sems):
        my_pos = lax.axis_index("i")

        barrier_sem = pltpu.get_barrier_semaphore()
        for d in range(1, N_DEV):
            peer = lax.rem(my_pos + d, N_DEV)
            pl.semaphore_signal(
                barrier_sem, inc=1,
                device_id=(peer,), device_id_type=pl.DeviceIdType.MESH,
            )
        pl.semaphore_wait(barrier_sem, N_DEV - 1)

        xv = x_ref[...]
        stats = jnp.concatenate(
            [jnp.sum(xv, axis=2), jnp.sum(xv * xv, axis=2)], axis=0
        )
        gather_buf[my_pos] = stats

        sends = []
        for d in range(1, N_DEV):
            peer = lax.rem(my_pos + d, N_DEV)
            rdma = pltpu.make_async_remote_copy(
                src_ref=gather_buf.at[my_pos],
                dst_ref=gather_buf.at[my_pos],
                send_sem=send_sems.at[d],
                recv_sem=recv_sems.at[my_pos],
                device_id=(peer,),
                device_id_type=pl.DeviceIdType.MESH,
            )
            rdma.start()
            sends.append(rdma)

        scale = jnp.dot(t_ref[...], wsc_ref[...],
                        preferred_element_type=jnp.float32)
        shift = jnp.dot(t_ref[...], wsh_ref[...],
                        preferred_element_type=jnp.float32)

        for d in range(1, N_DEV):
            src_pos = lax.rem(my_pos + d, N_DEV)
            recv = pltpu.make_async_remote_copy(
                src_ref=gather_buf.at[src_pos],
                dst_ref=gather_buf.at[src_pos],
                send_sem=send_sems.at[d],
                recv_sem=recv_sems.at[src_pos],
                device_id=(src_pos,),
                device_id_type=pl.DeviceIdType.MESH,
            )
            recv.wait_recv()
        for rdma in sends:
            rdma.wait_send()

        total = jnp.sum(gather_buf[...], axis=0)
        inv_c = 1.0 / (N_DEV * c_per)
        mean = total[0:b] * inv_c
        var = total[b:2 * b] * inv_c - mean * mean
        rstd = lax.rsqrt(var + 1e-5)
        h = (xv - mean[:, :, None]) * rstd[:, :, None]
        out_ref[...] = (h * (1.0 + scale)[:, None, :] + shift[:, None, :])

    return pl.pallas_call(
        body,
        out_shape=jax.ShapeDtypeStruct((b, s, c_per), jnp.float32),
        in_specs=[
            pl.BlockSpec(memory_space=pltpu.VMEM),
            pl.BlockSpec(memory_space=pltpu.VMEM),
            pl.BlockSpec(memory_space=pltpu.VMEM),
            pl.BlockSpec(memory_space=pltpu.VMEM),
        ],
        out_specs=pl.BlockSpec(memory_space=pltpu.VMEM),
        scratch_shapes=[
            pltpu.VMEM((N_DEV, 2 * b, s), jnp.float32),
            pltpu.SemaphoreType.DMA((N_DEV,)),
            pltpu.SemaphoreType.DMA((N_DEV,)),
        ],
        compiler_params=pltpu.CompilerParams(collective_id=0),
    )(x, t_emb, W_scale, W_shift)


# device time: 11740 ns/iter; 1.0024x vs baseline; 1.0024x over previous
import jax
import jax.numpy as jnp
from jax import lax
from jax.experimental import pallas as pl
from jax.experimental.pallas import tpu as pltpu

N_DEV = 8


def kernel(x, t_emb, W_scale, W_shift):
    b, s, c_per = x.shape

    def body(x_ref, t_ref, wsc_ref, wsh_ref, out_ref,
             stats_ref, gather_buf, send_sems, recv_sems):
        my_pos = lax.axis_index("i")

        xv = x_ref[...]
        stats_ref[...] = jnp.concatenate(
            [jnp.sum(xv, axis=2), jnp.sum(xv * xv, axis=2)], axis=0
        )
        scale = jnp.dot(t_ref[...], wsc_ref[...],
                        preferred_element_type=jnp.float32)
        shift = jnp.dot(t_ref[...], wsh_ref[...],
                        preferred_element_type=jnp.float32)

        barrier_sem = pltpu.get_barrier_semaphore()
        for d in range(1, N_DEV):
            peer = lax.rem(my_pos + d, N_DEV)
            pl.semaphore_signal(
                barrier_sem, inc=1,
                device_id=(peer,), device_id_type=pl.DeviceIdType.MESH,
            )
        pl.semaphore_wait(barrier_sem, N_DEV - 1)

        sends = []
        for d in range(1, N_DEV):
            peer = lax.rem(my_pos + d, N_DEV)
            rdma = pltpu.make_async_remote_copy(
                src_ref=stats_ref,
                dst_ref=gather_buf.at[N_DEV - 1 - d],
                send_sem=send_sems.at[d - 1],
                recv_sem=recv_sems.at[N_DEV - 1 - d],
                device_id=(peer,),
                device_id_type=pl.DeviceIdType.MESH,
            )
            rdma.start()
            sends.append(rdma)

        for slot in range(N_DEV - 1):
            recv = pltpu.make_async_remote_copy(
                src_ref=stats_ref,
                dst_ref=gather_buf.at[slot],
                send_sem=send_sems.at[slot],
                recv_sem=recv_sems.at[slot],
                device_id=(my_pos,),
                device_id_type=pl.DeviceIdType.MESH,
            )
            recv.wait_recv()
        for rdma in sends:
            rdma.wait_send()

        total = stats_ref[...] + jnp.sum(gather_buf[...], axis=0)
        inv_c = 1.0 / (N_DEV * c_per)
        mean = total[0:b] * inv_c
        var = total[b:2 * b] * inv_c - mean * mean
        rstd = lax.rsqrt(var + 1e-5)
        h = (xv - mean[:, :, None]) * rstd[:, :, None]
        out_ref[...] = (h * (1.0 + scale)[:, None, :] + shift[:, None, :])

    return pl.pallas_call(
        body,
        out_shape=jax.ShapeDtypeStruct((b, s, c_per), jnp.float32),
        in_specs=[
            pl.BlockSpec(memory_space=pltpu.VMEM),
            pl.BlockSpec(memory_space=pltpu.VMEM),
            pl.BlockSpec(memory_space=pltpu.VMEM),
            pl.BlockSpec(memory_space=pltpu.VMEM),
        ],
        out_specs=pl.BlockSpec(memory_space=pltpu.VMEM),
        scratch_shapes=[
            pltpu.VMEM((2 * b, s), jnp.float32),
            pltpu.VMEM((N_DEV - 1, 2 * b, s), jnp.float32),
            pltpu.SemaphoreType.DMA((N_DEV - 1,)),
            pltpu.SemaphoreType.DMA((N_DEV - 1,)),
        ],
        compiler_params=pltpu.CompilerParams(collective_id=0),
    )(x, t_emb, W_scale, W_shift)
